# Initial kernel scaffold; baseline (speedup 1.0000x reference)
import jax
import jax.numpy as jnp
from jax.experimental import pallas as pl


def kernel(new_xyz, xyz, gt_sdf):
    raise NotImplementedError("write your pallas kernel here")


# speedup vs baseline: 8.8594x; 8.8594x over previous
import functools

import numpy as np
import jax
import jax.numpy as jnp
from jax import lax
from jax.experimental import pallas as pl
from jax.experimental.pallas import tpu as pltpu
from jax.experimental.pallas import tpu_sc as plsc

_BI = 256
_CB = 1024
_NC, _NS, _L = 2, 16, 16
_NW = _NC * _NS
_G = 64


def _elastic_C() -> np.ndarray:
    vp = 0.4
    Ep = 0.21
    Ci = np.zeros((6, 6), dtype=np.float64)
    Ci[0, 0] = 1 / Ep
    Ci[0, 1] = -vp / Ep
    Ci[0, 2] = -vp / Ep
    Ci[1, 0] = -vp / Ep
    Ci[1, 1] = 1 / Ep
    Ci[1, 2] = -vp / Ep
    Ci[2, 0] = -vp
    Ci[2, 1] = -vp
    Ci[2, 2] = 1 / Ep
    Ci[3, 3] = 2 * (1 + vp) / Ep
    Ci[4, 4] = 2 * (1 + vp) / Ep
    Ci[5, 5] = 2 * (1 + vp) / Ep
    return np.linalg.inv(Ci).astype(np.float32)


_CMAT = _elastic_C()


def _lane_gather(x, idx):
    return lax.gather(
        x, idx[:, None],
        lax.GatherDimensionNumbers(offset_dims=(), collapsed_slice_dims=(0,),
                                   start_index_map=(0,)),
        (1,), mode=lax.GatherScatterMode.PROMISE_IN_BOUNDS)


def _run_compact_sc(comps, insidef, npad):
    C2 = npad // _NS
    steps = C2 // _L
    ng = C2 // _G
    mesh = plsc.VectorSubcoreMesh(core_axis_name="c", subcore_axis_name="s",
                                  num_cores=1)

    @functools.partial(
        pl.kernel,
        out_type=[jax.ShapeDtypeStruct((npad,), jnp.float32),
                  jax.ShapeDtypeStruct((_NS, _L), jnp.int32)],
        mesh=mesh,
        scratch_types=[pltpu.VMEM((C2,), jnp.float32),
                       pltpu.VMEM((C2 + 2 * _L,), jnp.float32),
                       pltpu.VMEM((1, _L), jnp.int32)],
    )
    def body_a1(ins_h, sidxf_h, cnts_h, insv, sidxf_v, cnt_v):
        sid = lax.axis_index("s")
        base = sid * C2
        pltpu.sync_copy(ins_h.at[pl.ds(base, C2)], insv)
        lane = lax.iota(jnp.int32, _L)
        wptr = jnp.int32(0)
        for k in range(steps):
            v = insv[pl.ds(k * _L, _L)]
            m = v > 0.0
            r = jnp.where(m, 1.0, 0.0)
            for d in (1, 2, 4, 8):
                g = _lane_gather(r, jnp.maximum(lane - d, 0))
                r = r + jnp.where(lane >= d, g, 0.0)
            ri = r.astype(jnp.int32)
            pop = ri[_L - 1]
            target = lane + 1
            lo = jnp.full((_L,), -1, jnp.int32)
            for sstep in (8, 4, 2, 1):
                mid = lo + sstep
                vm = _lane_gather(r, jnp.clip(mid, 0, _L - 1))
                ok = (mid <= _L - 1) & (vm.astype(jnp.int32) < target)
                lo = jnp.where(ok, mid, lo)
            perm = lo + 1
            gidxf = jnp.minimum(base + k * _L + perm, npad - 1).astype(jnp.float32)
            a8 = pl.multiple_of((wptr >> 3) << 3, 8)
            rotv = jnp.full((_L,), wptr - ((wptr >> 3) << 3), jnp.int32)
            old = sidxf_v[pl.ds(a8, _L)]
            hd = _lane_gather(gidxf, jnp.clip(lane - rotv, 0, _L - 1))
            sidxf_v[pl.ds(a8, _L)] = jnp.where(lane < rotv, old, hd)
            tl = _lane_gather(gidxf, jnp.clip(lane - rotv + _L, 0, _L - 1))
            sidxf_v[pl.ds(pl.multiple_of(a8 + _L, 8), _L)] = tl
            wptr = wptr + pop
        pltpu.sync_copy(sidxf_v.at[pl.ds(0, C2)], sidxf_h.at[pl.ds(base, C2)])
        cnt_v[0, :] = jnp.full((_L,), wptr, jnp.int32)
        pltpu.sync_copy(cnt_v, cnts_h.at[pl.ds(sid, 1)])

    sidxf, cnts = body_a1(insidef)
    cnt_s = cnts[:, 0]
    my16 = ((cnt_s + _L - 1) // _L) * _L
    off_s = jnp.concatenate([jnp.zeros((1,), my16.dtype),
                             jnp.cumsum(my16)[:-1]]).astype(jnp.int32)
    span = jnp.sum(my16, dtype=jnp.int32)
    offc = jnp.stack([off_s, cnt_s.astype(jnp.int32)] + [off_s] * (_L - 2),
                     axis=1)

    @functools.partial(
        pl.kernel,
        out_type=[jax.ShapeDtypeStruct((npad,), jnp.float32) for _ in range(7)],
        mesh=mesh,
        scratch_types=(
            [pltpu.VMEM((C2 + 2 * _L,), jnp.float32),
             pltpu.VMEM((C2,), jnp.int32)]
            + [pltpu.VMEM((C2,), jnp.float32) for _ in range(7)]
            + [pltpu.VMEM((1, _L), jnp.int32),
               pltpu.SemaphoreType.DMA]
        ),
    )
    def body_a2(nx_h, ny_h, nz_h, xx_h, xy_h, xz_h, sidxf_h, offc_h,
                cnx_h, cny_h, cnz_h, cxx_h, cxy_h, cxz_h, cal_h,
                sidxf_v, sidx_v, bnx, bny, bnz, bxx, bxy, bxz, bal,
                off_v, sem):
        sid = lax.axis_index("s")
        base = sid * C2
        lane = lax.iota(jnp.int32, _L)
        pltpu.sync_copy(offc_h.at[pl.ds(sid, 1)], off_v)
        row = off_v[0, :]
        off = row[0]
        count = row[1]
        pltpu.sync_copy(sidxf_h.at[pl.ds(base, C2)], sidxf_v.at[pl.ds(0, C2)])
        cntv = jnp.full((_L,), count, jnp.int32)
        for k in range(steps):
            sl = pl.ds(k * _L, _L)
            s = sidxf_v[sl]
            live = lane + k * _L < cntv
            s = jnp.where(live, jnp.clip(s, 0.0, float(npad - 1)), 0.0)
            sidx_v[sl] = s.astype(jnp.int32)
            bal[sl] = jnp.where(live, 1.0, 0.0)
        copies = []
        for g in range(ng):
            sl = pl.ds(g * _G, _G)
            for t, b in zip((nx_h, ny_h, nz_h, xx_h, xy_h, xz_h),
                            (bnx, bny, bnz, bxx, bxy, bxz)):
                copies.append(pltpu.async_copy(t.at[sidx_v.at[sl]], b.at[sl], sem))
        for cp in copies:
            cp.wait()
        my16w = ((count + _L - 1) >> 4) << 4
        obase = pl.multiple_of(off, _L)
        for k in range(steps):
            @pl.when(k * _L < my16w)
            def _():
                s2 = pl.ds(k * _L, _L)
                d2 = pl.ds(pl.multiple_of(obase + k * _L, _L), _L)
                for b, o in zip((bnx, bny, bnz, bxx, bxy, bxz, bal),
                                (cnx_h, cny_h, cnz_h, cxx_h, cxy_h, cxz_h, cal_h)):
                    pltpu.sync_copy(b.at[s2], o.at[d2])

    outs = body_a2(*comps, sidxf, offc)
    return tuple(outs) + (span[None],)


def _run_nn(spans, wrow, wcol):
    npad = wrow.shape[0]
    ncb = npad // _CB

    def body(spans_ref, wrow_ref, wcol_ref, idx_ref, val_ref, minv_ref):
        rb = pl.program_id(0)
        cb = pl.program_id(1)
        i0 = rb * _BI
        j0 = cb * _CB
        s0 = spans_ref[0]

        @pl.when(cb == 0)
        def _():
            minv_ref[...] = jnp.full((_BI, 1), jnp.inf, jnp.float32)
            idx_ref[...] = jnp.zeros((_BI, 1), jnp.int32)

        def compute(excl_self):
            wr = wrow_ref[...]
            wc = wcol_ref[...]
            xi, yi, zi = wr[:, 0:1], wr[:, 1:2], wr[:, 2:3]
            xj, yj, zj, mj = wc[0:1, :], wc[1:2, :], wc[2:3, :], wc[3:4, :]
            jid1 = lax.broadcasted_iota(jnp.int32, (1, _CB), 1) + j0
            colv = (jid1 < s0) & (mj > 0.0)
            dotv = jnp.dot(wr[:, 0:3], wc[0:3, :],
                           preferred_element_type=jnp.float32)
            sqj = (xj * xj + yj * yj) + zj * zj
            sqi = (xi * xi + yi * yi) + zi * zi
            d2 = (sqi + sqj) - 2.0 * dotv
            d2 = jnp.maximum(d2, 0.0)
            s = jnp.where(colv, d2, jnp.inf)
            jids = lax.broadcasted_iota(jnp.int32, s.shape, 1) + j0
            if excl_self:
                iids = lax.broadcasted_iota(jnp.int32, s.shape, 0) + i0
                s = jnp.where(jids == iids, jnp.inf, s)
            cmin = jnp.min(s, axis=1, keepdims=True)
            cidx = jnp.min(jnp.where(s == cmin, jids, jnp.int32(2**30)),
                           axis=1, keepdims=True)
            cur = minv_ref[...]
            upd = cmin < cur
            idx_ref[...] = jnp.where(upd, cidx, idx_ref[...])
            minv_ref[...] = jnp.where(upd, cmin, cur)

        active = (i0 < s0) & (j0 < s0)
        on_diag = (i0 < j0 + _CB) & (j0 < i0 + _BI)

        @pl.when(active & on_diag)
        def _():
            compute(True)

        @pl.when(active & jnp.logical_not(on_diag))
        def _():
            compute(False)

        @pl.when(cb == ncb - 1)
        def _():
            wr = wrow_ref[...]
            mi = wr[:, 3:4]
            iri = lax.broadcasted_iota(jnp.int32, (_BI, 1), 0) + i0
            rv = iri < s0
            nnd = jnp.sqrt(minv_ref[...])
            val_ref[...] = (rv & (mi > 0.0) & (nnd > 1e-8)).astype(jnp.float32)

    return pl.pallas_call(
        body,
        grid=(npad // _BI, ncb),
        in_specs=[
            pl.BlockSpec(memory_space=pltpu.SMEM),
            pl.BlockSpec((_BI, 4), lambda r, c: (r, 0)),
            pl.BlockSpec((4, _CB), lambda r, c: (0, c)),
        ],
        out_specs=[
            pl.BlockSpec((_BI, 1), lambda r, c: (r, 0)),
            pl.BlockSpec((_BI, 1), lambda r, c: (r, 0)),
        ],
        out_shape=[
            jax.ShapeDtypeStruct((npad, 1), jnp.int32),
            jax.ShapeDtypeStruct((npad, 1), jnp.float32),
        ],
        scratch_shapes=[pltpu.VMEM((_BI, 1), jnp.float32)],
    )(spans, wrow, wcol)


def _run_strain_sc(comps, nn_idx, validf):
    npad = comps[0].shape[0]
    C = npad // _NW
    steps = C // _L
    ngroups = C // _G
    cm = [[float(_CMAT[a, b]) for b in range(6)] for a in range(6)]
    mesh = plsc.VectorSubcoreMesh(core_axis_name="c", subcore_axis_name="s")

    @functools.partial(
        pl.kernel,
        out_type=jax.ShapeDtypeStruct((_NW, 2, _L), jnp.float32),
        mesh=mesh,
        scratch_types=(
            [pltpu.VMEM((C,), jnp.int32),
             pltpu.VMEM((C,), jnp.float32)]
            + [pltpu.VMEM((C,), jnp.float32) for _ in range(12)]
            + [pltpu.VMEM((2, _L), jnp.float32),
               pltpu.SemaphoreType.DMA]
        ),
    )
    def body(nx_h, ny_h, nz_h, xx_h, xy_h, xz_h, idx_hbm, val_hbm, out_hbm,
             idx_v, val_v,
             onx_v, ony_v, onz_v, oxx_v, oxy_v, oxz_v,
             gnx_v, gny_v, gnz_v, gxx_v, gxy_v, gxz_v,
             acc_v, sem):
        wid = lax.axis_index("c") * _NS + lax.axis_index("s")
        base = wid * C
        tabs = (nx_h, ny_h, nz_h, xx_h, xy_h, xz_h)
        owns = (onx_v, ony_v, onz_v, oxx_v, oxy_v, oxz_v)
        nbrs = (gnx_v, gny_v, gnz_v, gxx_v, gxy_v, gxz_v)
        pltpu.sync_copy(idx_hbm.at[pl.ds(base, C)], idx_v)
        pltpu.sync_copy(val_hbm.at[pl.ds(base, C)], val_v)
        for t, o in zip(tabs, owns):
            pltpu.sync_copy(t.at[pl.ds(base, C)], o)
        copies = []
        for g in range(ngroups):
            sl = pl.ds(g * _G, _G)
            for t, b in zip(tabs, nbrs):
                copies.append(pltpu.async_copy(t.at[idx_v.at[sl]], b.at[sl], sem))
        for cp in copies:
            cp.wait()

        qacc = jnp.zeros((_L,), jnp.float32)
        cacc = jnp.zeros((_L,), jnp.float32)
        for k in range(steps):
            sl = pl.ds(k * _L, _L)
            onx, ony, onz = onx_v[sl], ony_v[sl], onz_v[sl]
            oxx, oxy, oxz = oxx_v[sl], oxy_v[sl], oxz_v[sl]
            gnx, gny, gnz = gnx_v[sl], gny_v[sl], gnz_v[sl]
            gxx, gxy, gxz = gxx_v[sl], gxy_v[sl], gxz_v[sl]
            du = (gnx - gxx) - (onx - oxx)
            dv = (gny - gxy) - (ony - oxy)
            dw = (gnz - gxz) - (onz - oxz)
            dx = gnx - onx + 1e-8
            dy = gny - ony + 1e-8
            dz = gnz - onz + 1e-8
            e = (du / dx, dv / dy, dw / dz,
                 (du / dy + dv / dx) * 0.5,
                 (du / dz + dw / dx) * 0.5,
                 (dw / dy + dv / dz) * 0.5)
            q = jnp.zeros((_L,), jnp.float32)
            for a in range(6):
                fa = cm[a][0] * e[0]
                for b in range(1, 6):
                    fa = fa + cm[a][b] * e[b]
                q = q + e[a] * fa
            v = val_v[sl]
            q = jnp.where(v > 0.0, q, 0.0)
            qacc = qacc + q * q
            cacc = cacc + v
        acc_v[0, :] = qacc
        acc_v[1, :] = cacc
        pltpu.sync_copy(acc_v, out_hbm.at[wid])

    return body(*comps, nn_idx, validf)


def kernel(new_xyz, xyz, gt_sdf):
    n = new_xyz.shape[0]
    blk = _NW * _G
    npad = ((n + blk - 1) // blk) * blk
    inside = (gt_sdf < 1e-8).astype(jnp.float32)
    zpad = jnp.zeros((npad - n,), jnp.float32)
    comps = tuple(jnp.concatenate([new_xyz[:, c], zpad]) for c in range(3)) + \
            tuple(jnp.concatenate([xyz[:, c], zpad]) for c in range(3))
    insidef = jnp.concatenate([inside, zpad])
    cnx, cny, cnz, cxx, cxy, cxz, cal, spans = _run_compact_sc(comps, insidef, npad)
    wcol = jnp.stack([cnx, cny, cnz, cal])
    idx2, val2 = _run_nn(spans, wcol.T, wcol)
    parts = _run_strain_sc((cnx, cny, cnz, cxx, cxy, cxz), idx2[:, 0], val2[:, 0])
    qsq = jnp.sum(parts[:, 0, :])
    cnt = jnp.sum(parts[:, 1, :])
    return jnp.sqrt(qsq) / cnt
